# baseline (speedup 1.0000x reference)
import functools

import jax
import jax.numpy as jnp
from jax import lax
from jax.experimental import pallas as pl
from jax.experimental.pallas import tpu as pltpu
from jax.experimental.pallas import tpu_sc as plsc

NUM_CORES = 2
NUM_SUBCORES = 16
NW = NUM_CORES * NUM_SUBCORES

BATCH = 4096
HIST = 200
D = 64
DP = 128
ROWS_W = BATCH // NW
STEPS = 2 * ROWS_W
SPLIT = 96
SZ = (SPLIT, HIST - SPLIT)
NBUF = 4

_mesh = plsc.VectorSubcoreMesh(
    core_axis_name="c", subcore_axis_name="s",
    num_cores=NUM_CORES, num_subcores=NUM_SUBCORES)


@functools.partial(
    pl.kernel,
    out_type=jax.ShapeDtypeStruct((BATCH * HIST, DP), jnp.float32),
    mesh=_mesh,
    scratch_types=[
        pltpu.VMEM((ROWS_W, HIST), jnp.int32),
        [pltpu.VMEM((HIST - SPLIT, DP), jnp.float32) for _ in range(NBUF)],
        [pltpu.SemaphoreType.DMA for _ in range(NBUF)],
        [pltpu.SemaphoreType.DMA for _ in range(NBUF)],
    ],
    compiler_params=pltpu.CompilerParams(
        use_tc_tiling_on_sc=False, skip_device_barrier=True),
)
def _gather_kernel(table_hbm, idx_hbm, out_hbm, idx_v, rows, gsem, wsem):
    wid = lax.axis_index("s") * NUM_CORES + lax.axis_index("c")
    row0 = wid * ROWS_W
    pltpu.sync_copy(idx_hbm.at[pl.ds(row0, ROWS_W)], idx_v)

    def fire_gather(s, k, b):
        r, h = s // 2, k % 2
        pltpu.make_async_copy(
            table_hbm.at[idx_v.at[r, pl.ds(h * SPLIT, SZ[h])]],
            rows[b].at[pl.ds(0, SZ[h])], gsem[b]).start()

    def wait_gather(k, b):
        h = k % 2
        pltpu.make_async_copy(
            table_hbm.at[idx_v.at[0, pl.ds(0, SZ[h])]],
            rows[b].at[pl.ds(0, SZ[h])], gsem[b]).wait()

    def fire_write(s, k, b):
        r, h = s // 2, k % 2
        pltpu.make_async_copy(
            rows[b].at[pl.ds(0, SZ[h])],
            out_hbm.at[pl.ds((row0 + r) * HIST + h * SPLIT, SZ[h])],
            wsem[b]).start()

    def wait_write(k, b):
        h = k % 2
        pltpu.make_async_copy(
            rows[b].at[pl.ds(0, SZ[h])],
            out_hbm.at[pl.ds(0, SZ[h])], wsem[b]).wait()

    for s0 in range(NBUF - 1):
        fire_gather(s0, s0, s0)

    @pl.loop(0, STEPS, step=NBUF)
    def _grp(g):
        for k in range(NBUF):
            s = g + k
            bg = (k + NBUF - 1) % NBUF
            kg = (k + NBUF - 1) % NBUF

            @pl.when(s >= 1)
            def _():
                wait_write(kg, bg)

            @pl.when(s + NBUF - 1 < STEPS)
            def _():
                fire_gather(s + NBUF - 1, kg, bg)

            wait_gather(k, k)
            fire_write(s, k, k)

    wait_write((STEPS - 1) % NBUF, (STEPS - 1) % NBUF)


NUM_EMB = 1000000
NTILE = 7813
NFULL = NTILE - 1
K1_STEPS = 245


@functools.partial(
    pl.kernel,
    out_type=jax.ShapeDtypeStruct((125000, 8, DP), jnp.float32),
    mesh=_mesh,
    scratch_types=[
        [pltpu.VMEM((8, 8, DP), jnp.float32) for _ in range(2)],
        [pltpu.VMEM((16, 8, DP), jnp.float32) for _ in range(2)],
        [pltpu.SemaphoreType.DMA for _ in range(2)],
        [pltpu.SemaphoreType.DMA for _ in range(2)],
    ],
    compiler_params=pltpu.CompilerParams(
        use_tc_tiling_on_sc=True, skip_device_barrier=True,
        needs_layout_passes=False),
)
def _relayout_kernel(wt_hbm, wtail_hbm, wp_hbm, vbufs, tbufs, rsem, wsem):
    wid = lax.axis_index("s") * NUM_CORES + lax.axis_index("c")
    nj = 244 + (wid < 4).astype(jnp.int32)
    iota = lax.iota(jnp.int32, 16)
    g_ids = [(16 * k + iota) >> 3 for k in range(4)]
    s_ids = [(16 * k + iota) & 7 for k in range(4)]

    def fire_read(c, b):
        for g in range(8):
            pltpu.make_async_copy(
                wt_hbm.at[g, :, pl.ds(c * 128, 128)],
                vbufs[b].at[g], rsem[b]).start()

    def wait_read(b):
        for g in range(8):
            pltpu.make_async_copy(
                wt_hbm.at[g, :, pl.ds(0, 128)],
                vbufs[b].at[g], rsem[b]).wait()

    def transpose(b, nm):
        @pl.loop(0, nm, unroll=8)
        def _m(m):
            t, s2 = m >> 3, m & 7
            m_ids = jnp.full((16,), 0, jnp.int32) + m
            for k in range(4):
                v = plsc.load_gather(vbufs[b], [g_ids[k], s_ids[k], m_ids])
                tbufs[b][t, s2, pl.ds(16 * k, 16)] = v

    def fire_write(c, b):
        pltpu.make_async_copy(
            tbufs[b], wp_hbm.at[pl.ds(16 * c, 16)], wsem[b]).start()

    def wait_write(b):
        pltpu.make_async_copy(
            tbufs[b], wp_hbm.at[pl.ds(0, 16)], wsem[b]).wait()

    fire_read(wid, 0)

    @pl.loop(0, K1_STEPS + 1, step=2)
    def _blk(jj):
        for k in range(2):
            j = jj + k

            @pl.when(j + 1 < nj)
            def _():
                fire_read(wid + 32 * (j + 1), (k + 1) % 2)

            @pl.when(j < nj)
            def _():
                wait_read(k)

                @pl.when(j >= 2)
                def _():
                    wait_write(k)

                transpose(k, 128)
                fire_write(wid + 32 * j, k)

    wait_write(0)
    wait_write(1)

    @pl.when(wid == 4)
    def _tail():
        pltpu.sync_copy(wtail_hbm, wp_hbm.at[pl.ds(16 * NFULL, 8)])


def kernel(input_, weight):
    wt = weight.T.reshape(8, 8, NUM_EMB)
    wtail = jnp.pad(weight[128 * NFULL:], ((0, 0), (0, DP - D)))
    wp = _relayout_kernel(wt, wtail.reshape(8, 8, DP)).reshape(NUM_EMB, DP)
    outp = _gather_kernel(wp, input_)
    return outp[:, :D].reshape(BATCH, HIST, D)


# speedup vs baseline: 1.9547x; 1.9547x over previous
import functools

import jax
import jax.numpy as jnp
from jax import lax
from jax.experimental import pallas as pl
from jax.experimental.pallas import tpu as pltpu
from jax.experimental.pallas import tpu_sc as plsc

NUM_CORES = 2
NUM_SUBCORES = 16
NW = NUM_CORES * NUM_SUBCORES

BATCH = 4096
HIST = 200
D = 64
DP = 128
ROWS_W = BATCH // NW
STEPS = 2 * ROWS_W
SPLIT = 96
SZ = (SPLIT, HIST - SPLIT)
NBUF = 4

_mesh = plsc.VectorSubcoreMesh(
    core_axis_name="c", subcore_axis_name="s",
    num_cores=NUM_CORES, num_subcores=NUM_SUBCORES)


@functools.partial(
    pl.kernel,
    out_type=jax.ShapeDtypeStruct((BATCH * HIST, DP), jnp.float32),
    mesh=_mesh,
    scratch_types=[
        pltpu.VMEM((ROWS_W, HIST), jnp.int32),
        [pltpu.VMEM((HIST - SPLIT, DP), jnp.float32) for _ in range(NBUF)],
        [pltpu.SemaphoreType.DMA for _ in range(NBUF)],
        [pltpu.SemaphoreType.DMA for _ in range(NBUF)],
    ],
    compiler_params=pltpu.CompilerParams(
        use_tc_tiling_on_sc=False, skip_device_barrier=True),
)
def _gather_kernel(table_hbm, idx_hbm, out_hbm, idx_v, rows, gsem, wsem):
    wid = lax.axis_index("s") * NUM_CORES + lax.axis_index("c")
    row0 = wid * ROWS_W
    pltpu.sync_copy(idx_hbm.at[pl.ds(row0, ROWS_W)], idx_v)

    def fire_gather(s, k, b):
        r, h = s // 2, k % 2
        pltpu.make_async_copy(
            table_hbm.at[idx_v.at[r, pl.ds(h * SPLIT, SZ[h])]],
            rows[b].at[pl.ds(0, SZ[h])], gsem[b]).start()

    def wait_gather(k, b):
        h = k % 2
        pltpu.make_async_copy(
            table_hbm.at[idx_v.at[0, pl.ds(0, SZ[h])]],
            rows[b].at[pl.ds(0, SZ[h])], gsem[b]).wait()

    def fire_write(s, k, b):
        r, h = s // 2, k % 2
        pltpu.make_async_copy(
            rows[b].at[pl.ds(0, SZ[h])],
            out_hbm.at[pl.ds((row0 + r) * HIST + h * SPLIT, SZ[h])],
            wsem[b]).start()

    def wait_write(k, b):
        h = k % 2
        pltpu.make_async_copy(
            rows[b].at[pl.ds(0, SZ[h])],
            out_hbm.at[pl.ds(0, SZ[h])], wsem[b]).wait()

    for s0 in range(NBUF - 1):
        fire_gather(s0, s0, s0)

    @pl.loop(0, STEPS, step=NBUF)
    def _grp(g):
        for k in range(NBUF):
            s = g + k
            bg = (k + NBUF - 1) % NBUF
            kg = (k + NBUF - 1) % NBUF

            @pl.when(s >= 1)
            def _():
                wait_write(kg, bg)

            @pl.when(s + NBUF - 1 < STEPS)
            def _():
                fire_gather(s + NBUF - 1, kg, bg)

            wait_gather(k, k)
            fire_write(s, k, k)

    wait_write((STEPS - 1) % NBUF, (STEPS - 1) % NBUF)


def kernel(input_, weight):
    wp = jnp.pad(weight, ((0, 0), (0, DP - D)))
    outp = _gather_kernel(wp, input_)
    return outp[:, :D].reshape(BATCH, HIST, D)
